# baseline (speedup 1.0000x reference)
import functools

import jax
import jax.numpy as jnp
from jax import lax
from jax.experimental import pallas as pl
from jax.experimental.pallas import tpu as pltpu
from jax.experimental.pallas import tpu_sc as plsc

H = 128
N = 10000
NPAD = 10240
NC, NS = 2, 16
CHUNK = 88
KCH = 116
PT = KCH * CHUNK
RPS = NPAD // NS


def _sc_body(x_hbm, ed_hbm, w_hbm, msg_hbm, deg_hbm,
             ed0, ed1, ed2, ed3, wx0, wx1, gb0, gb1,
             se0, se1, se2, se3, sw0, sw1, sg0, sg1,
             ss0, ss1, ss2, ss3, acc):
    c = lax.axis_index("c")
    s = lax.axis_index("s")
    eds = (ed0, ed1, ed2, ed3)
    wxs = (wx0, wx1)
    gbs = (gb0, gb1)
    ses = (se0, se1, se2, se3)
    sws = (sw0, sw1)
    sgs = (sg0, sg1)
    sss = (ss0, ss1, ss2, ss3)

    def zrow(i, carry):
        for t in range(H // 16):
            gb1[i, pl.ds(16 * t, 16)] = jnp.zeros((16,), jnp.float32)
        return carry
    lax.fori_loop(0, CHUNK, zrow, 0)
    base = s * RPS
    def zcp(k, carry):
        pltpu.sync_copy(gb1.at[pl.ds(0, 64)], acc.at[pl.ds(base + k * 64, 64)])
        return carry
    lax.fori_loop(0, RPS // 64, zcp, 0)
    plsc.subcore_barrier()

    pltpu.sync_copy(ed_hbm.at[c, s, 0], ed0)
    pltpu.sync_copy(w_hbm.at[c, s, 0], wx0)
    pltpu.async_copy(x_hbm.at[ed0.at[0]], gb0, sg0)

    def phase1(jj, carry):
        for p in range(2):
            j = jj * 2 + p
            q = 1 - p
            jn = j + 1
            @pl.when(jn < KCH)
            def _():
                pltpu.async_copy(ed_hbm.at[c, s, jn], eds[q], ses[q])
                pltpu.async_copy(w_hbm.at[c, s, jn], wxs[q], sws[q])
            pltpu.make_async_copy(x_hbm.at[eds[p].at[0]], gbs[p],
                                  sgs[p]).wait()

            def row(i, rc):
                wv = wxs[p][i, :]
                for t in range(H // 16):
                    gbs[p][i, pl.ds(16 * t, 16)] = (
                        gbs[p][i, pl.ds(16 * t, 16)] * wv)
                return rc
            lax.fori_loop(0, CHUNK, row, 0, unroll=4)

            @pl.when(jn < KCH)
            def _():
                pltpu.make_async_copy(ed_hbm.at[c, s, jn], eds[q],
                                      ses[q]).wait()
                pltpu.make_async_copy(w_hbm.at[c, s, jn], wxs[q],
                                      sws[q]).wait()
                pltpu.async_copy(x_hbm.at[eds[q].at[0]], gbs[q], sgs[q])
            pltpu.sync_copy(gbs[p], acc.at[eds[p].at[1]], add=True)
        return carry
    lax.fori_loop(0, KCH // 2, phase1, 0)
    plsc.subcore_barrier()

    def zrow2(i, carry):
        for t in range(H // 16):
            gb1[i, pl.ds(16 * t, 16)] = jnp.zeros((16,), jnp.float32)
        return carry
    lax.fori_loop(0, CHUNK, zrow2, 0)
    def wb(k, carry):
        r = base + k * 64
        pltpu.sync_copy(acc.at[pl.ds(r, 64)], gb0.at[pl.ds(0, 64)])
        pltpu.sync_copy(gb0.at[pl.ds(0, 64)], msg_hbm.at[c, pl.ds(r, 64)])
        pltpu.sync_copy(gb1.at[pl.ds(0, 64)], acc.at[pl.ds(r, 64)])
        return carry
    lax.fori_loop(0, RPS // 64, wb, 0)

    def orow(i, carry):
        for t in range(H // 16):
            gb0[i, pl.ds(16 * t, 16)] = jnp.ones((16,), jnp.float32)
        return carry
    lax.fori_loop(0, CHUNK, orow, 0)
    plsc.subcore_barrier()

    pltpu.sync_copy(ed_hbm.at[c, s, 0], ed0)
    def phase2(jj, carry):
        for p in range(2):
            j = jj * 2 + p
            q = 1 - p
            jn = j + 1
            @pl.when(jn < KCH)
            def _():
                pltpu.async_copy(ed_hbm.at[c, s, jn], eds[q], ses[q])
            @pl.when(j > 0)
            def _():
                pltpu.make_async_copy(ed_hbm.at[c, s, 0], eds[p],
                                      ses[p]).wait()
            pltpu.sync_copy(gb0, acc.at[eds[p].at[1]], add=True)
        return carry
    lax.fori_loop(0, KCH // 2, phase2, 0)
    plsc.subcore_barrier()

    def wbd(k, carry):
        r = base + k * 64
        pltpu.sync_copy(acc.at[pl.ds(r, 64)], gb1.at[pl.ds(0, 64)])
        pltpu.sync_copy(gb1.at[pl.ds(0, 64)], deg_hbm.at[c, pl.ds(r, 64)])
        return carry
    lax.fori_loop(0, RPS // 64, wbd, 0)


_sc_aggregate = functools.partial(
    pl.kernel,
    mesh=plsc.VectorSubcoreMesh(core_axis_name="c", subcore_axis_name="s"),
    out_type=[
        jax.ShapeDtypeStruct((NC, NPAD, H), jnp.float32),
        jax.ShapeDtypeStruct((NC, NPAD, H), jnp.float32),
    ],
    scratch_types=(
        [pltpu.VMEM((2, CHUNK), jnp.int32) for _ in range(4)]
        + [pltpu.VMEM((CHUNK, 16), jnp.float32) for _ in range(2)]
        + [pltpu.VMEM((CHUNK, H), jnp.float32) for _ in range(2)]
        + [pltpu.SemaphoreType.DMA for _ in range(12)]
        + [pltpu.VMEM_SHARED((NPAD, H), jnp.float32)]
    ),
)(_sc_body)


def _tc_body(x_ref, m_ref, d_ref, wz_ref, wr_ref, wh_ref,
             bz_ref, br_ref, bh_ref, g_ref, bt_ref, o_ref):
    m = m_ref[0] + m_ref[1]
    deg = d_ref[0, :, 0:1] + d_ref[1, :, 0:1]
    deg = jnp.maximum(deg, 1.0)
    msg = m / deg
    xb = x_ref[...]

    def lin(wref, bref, a, b):
        w = wref[...]
        return (jnp.dot(a, w[:H], preferred_element_type=jnp.float32)
                + jnp.dot(b, w[H:], preferred_element_type=jnp.float32)
                + bref[...])

    z = jax.nn.sigmoid(lin(wz_ref, bz_ref, xb, msg))
    r = jax.nn.sigmoid(lin(wr_ref, br_ref, xb, msg))
    ht = jnp.tanh(lin(wh_ref, bh_ref, r * xb, msg))
    xn = (1.0 - z) * xb + z * ht
    mu = jnp.mean(xn, axis=-1, keepdims=True)
    var = jnp.mean((xn - mu) ** 2, axis=-1, keepdims=True)
    o_ref[...] = (xn - mu) * lax.rsqrt(var + 1e-5) * g_ref[...] + bt_ref[...]


def _tc_gru(x, msgp, degp, W_z, W_r, W_h, b_z, b_r, b_h, g, b):
    B = 2000
    grid = (N // B,)
    full = lambda shape: pl.BlockSpec(shape, lambda i: (0,) * len(shape))
    return pl.pallas_call(
        _tc_body,
        grid=grid,
        in_specs=[
            pl.BlockSpec((B, H), lambda i: (i, 0)),
            pl.BlockSpec((NC, B, H), lambda i: (0, i, 0)),
            pl.BlockSpec((NC, B, H), lambda i: (0, i, 0)),
            full((2 * H, H)), full((2 * H, H)), full((2 * H, H)),
            full((1, H)), full((1, H)), full((1, H)),
            full((1, H)), full((1, H)),
        ],
        out_specs=pl.BlockSpec((B, H), lambda i: (i, 0)),
        out_shape=jax.ShapeDtypeStruct((N, H), jnp.float32),
    )(x, msgp, degp, W_z, W_r, W_h, b_z, b_r, b_h, g, b)


def kernel(x, edge_index, edge_weights, W_z, b_z, W_r, b_r, W_h, b_h,
           ln_gamma, ln_beta):
    src = edge_index[0].astype(jnp.int32)
    dst = edge_index[1].astype(jnp.int32)
    E = src.shape[0]
    EP = NC * NS * PT
    pad = EP - E
    src_p = jnp.concatenate([src, jnp.zeros((pad,), jnp.int32)])
    dst_p = jnp.concatenate([dst, jnp.full((pad,), N, jnp.int32)])
    w_p = jnp.concatenate([edge_weights.astype(jnp.float32),
                           jnp.zeros((pad,), jnp.float32)])
    ed_t = jnp.stack([src_p.reshape(NC, NS, KCH, CHUNK),
                      dst_p.reshape(NC, NS, KCH, CHUNK)], axis=3)
    w_t = jnp.broadcast_to(w_p[:, None], (EP, 16)).reshape(NC, NS, KCH, CHUNK, 16)
    msgp, degp = _sc_aggregate(x, ed_t, w_t)
    r2 = lambda v: v.reshape(1, H)
    return _tc_gru(x, msgp, degp, W_z, W_r, W_h,
                   r2(b_z), r2(b_r), r2(b_h), r2(ln_gamma), r2(ln_beta))


# speedup vs baseline: 1.6109x; 1.6109x over previous
import functools

import jax
import jax.numpy as jnp
from jax import lax
from jax.experimental import pallas as pl
from jax.experimental.pallas import tpu as pltpu
from jax.experimental.pallas import tpu_sc as plsc

H = 128
N = 10000
NPAD = 10240
NC, NS = 2, 16
CHUNK = 88
KCH = 114
PT = KCH * CHUNK
RPS = NPAD // NS


def _sc_body(x_hbm, ed_hbm, w_hbm, msg_hbm, deg_hbm,
             ed0, ed1, wx0, wx1, gb0, gb1,
             se0, se1, sw0, sw1, sg0, sg1, acc):
    c = lax.axis_index("c")
    s = lax.axis_index("s")
    eds = (ed0, ed1)
    wxs = (wx0, wx1)
    gbs = (gb0, gb1)
    ses = (se0, se1)
    sws = (sw0, sw1)
    sgs = (sg0, sg1)

    def zrow(i, carry):
        for t in range(H // 16):
            gb1[i, pl.ds(16 * t, 16)] = jnp.zeros((16,), jnp.float32)
        return carry
    lax.fori_loop(0, CHUNK, zrow, 0)
    base = s * RPS
    def zcp(k, carry):
        pltpu.sync_copy(gb1.at[pl.ds(0, 64)], acc.at[pl.ds(base + k * 64, 64)])
        return carry
    lax.fori_loop(0, RPS // 64, zcp, 0)
    plsc.subcore_barrier()

    pltpu.sync_copy(ed_hbm.at[c, s, 0], ed0)
    pltpu.sync_copy(w_hbm.at[c, s, 0], wx0)
    pltpu.async_copy(x_hbm.at[ed0.at[0]], gb0, sg0)

    def phase1(jj, carry):
        for p in range(2):
            j = jj * 2 + p
            q = 1 - p
            jn = j + 1
            @pl.when(jn < KCH)
            def _():
                pltpu.async_copy(ed_hbm.at[c, s, jn], eds[q], ses[q])
                pltpu.async_copy(w_hbm.at[c, s, jn], wxs[q], sws[q])
            pltpu.make_async_copy(x_hbm.at[eds[p].at[0]], gbs[p],
                                  sgs[p]).wait()

            def row(i, rc):
                wv = wxs[p][i, :]
                for t in range(H // 16):
                    gbs[p][i, pl.ds(16 * t, 16)] = (
                        gbs[p][i, pl.ds(16 * t, 16)] * wv)
                return rc
            lax.fori_loop(0, CHUNK, row, 0, unroll=4)

            @pl.when(jn < KCH)
            def _():
                pltpu.make_async_copy(ed_hbm.at[c, s, jn], eds[q],
                                      ses[q]).wait()
                pltpu.make_async_copy(w_hbm.at[c, s, jn], wxs[q],
                                      sws[q]).wait()
                pltpu.async_copy(x_hbm.at[eds[q].at[0]], gbs[q], sgs[q])
            pltpu.sync_copy(gbs[p], acc.at[eds[p].at[1]], add=True)
        return carry
    lax.fori_loop(0, KCH // 2, phase1, 0)
    plsc.subcore_barrier()

    def zrow2(i, carry):
        for t in range(H // 16):
            gb1[i, pl.ds(16 * t, 16)] = jnp.zeros((16,), jnp.float32)
        return carry
    lax.fori_loop(0, CHUNK, zrow2, 0)
    def wb(k, carry):
        r = base + k * 64
        pltpu.sync_copy(acc.at[pl.ds(r, 64)], gb0.at[pl.ds(0, 64)])
        pltpu.sync_copy(gb0.at[pl.ds(0, 64)], msg_hbm.at[c, pl.ds(r, 64)])
        pltpu.sync_copy(gb1.at[pl.ds(0, 64)], acc.at[pl.ds(r, 64)])
        return carry
    lax.fori_loop(0, RPS // 64, wb, 0)

    def orow(i, carry):
        for t in range(H // 16):
            gb0[i, pl.ds(16 * t, 16)] = jnp.ones((16,), jnp.float32)
        return carry
    lax.fori_loop(0, CHUNK, orow, 0)
    plsc.subcore_barrier()

    pltpu.sync_copy(ed_hbm.at[c, s, 0], ed0)
    def phase2(jj, carry):
        for p in range(2):
            j = jj * 2 + p
            q = 1 - p
            jn = j + 1
            @pl.when(jn < KCH)
            def _():
                pltpu.async_copy(ed_hbm.at[c, s, jn], eds[q], ses[q])
            @pl.when(j > 0)
            def _():
                pltpu.make_async_copy(ed_hbm.at[c, s, 0], eds[p],
                                      ses[p]).wait()
            pltpu.sync_copy(gb0, acc.at[eds[p].at[1]], add=True)
        return carry
    lax.fori_loop(0, KCH // 2, phase2, 0)
    plsc.subcore_barrier()

    def wbd(k, carry):
        r = base + k * 64
        pltpu.sync_copy(acc.at[pl.ds(r, 64)], gb1.at[pl.ds(0, 64)])
        pltpu.sync_copy(gb1.at[pl.ds(0, 64)], deg_hbm.at[c, pl.ds(r, 64)])
        return carry
    lax.fori_loop(0, RPS // 64, wbd, 0)


_sc_aggregate = functools.partial(
    pl.kernel,
    mesh=plsc.VectorSubcoreMesh(core_axis_name="c", subcore_axis_name="s"),
    out_type=[
        jax.ShapeDtypeStruct((NC, NPAD, H), jnp.float32),
        jax.ShapeDtypeStruct((NC, NPAD, H), jnp.float32),
    ],
    scratch_types=(
        [pltpu.VMEM((2, CHUNK), jnp.int32) for _ in range(2)]
        + [pltpu.VMEM((CHUNK, 16), jnp.float32) for _ in range(2)]
        + [pltpu.VMEM((CHUNK, H), jnp.float32) for _ in range(2)]
        + [pltpu.SemaphoreType.DMA for _ in range(6)]
        + [pltpu.VMEM_SHARED((NPAD, H), jnp.float32)]
    ),
)(_sc_body)


def _tc_body(x_ref, m_ref, d_ref, wz_ref, wr_ref, wh_ref,
             bz_ref, br_ref, bh_ref, g_ref, bt_ref, o_ref):
    m = m_ref[0] + m_ref[1]
    deg = d_ref[0, :, 0:1] + d_ref[1, :, 0:1]
    deg = jnp.maximum(deg, 1.0)
    msg = m / deg
    xb = x_ref[...]

    def lin(wref, bref, a, b):
        w = wref[...]
        return (jnp.dot(a, w[:H], preferred_element_type=jnp.float32)
                + jnp.dot(b, w[H:], preferred_element_type=jnp.float32)
                + bref[...])

    z = jax.nn.sigmoid(lin(wz_ref, bz_ref, xb, msg))
    r = jax.nn.sigmoid(lin(wr_ref, br_ref, xb, msg))
    ht = jnp.tanh(lin(wh_ref, bh_ref, r * xb, msg))
    xn = (1.0 - z) * xb + z * ht
    mu = jnp.mean(xn, axis=-1, keepdims=True)
    var = jnp.mean((xn - mu) ** 2, axis=-1, keepdims=True)
    o_ref[...] = (xn - mu) * lax.rsqrt(var + 1e-5) * g_ref[...] + bt_ref[...]


def _tc_gru(x, msgp, degp, W_z, W_r, W_h, b_z, b_r, b_h, g, b):
    B = 2000
    grid = (N // B,)
    full = lambda shape: pl.BlockSpec(shape, lambda i: (0,) * len(shape))
    return pl.pallas_call(
        _tc_body,
        grid=grid,
        in_specs=[
            pl.BlockSpec((B, H), lambda i: (i, 0)),
            pl.BlockSpec((NC, B, H), lambda i: (0, i, 0)),
            pl.BlockSpec((NC, B, H), lambda i: (0, i, 0)),
            full((2 * H, H)), full((2 * H, H)), full((2 * H, H)),
            full((1, H)), full((1, H)), full((1, H)),
            full((1, H)), full((1, H)),
        ],
        out_specs=pl.BlockSpec((B, H), lambda i: (i, 0)),
        out_shape=jax.ShapeDtypeStruct((N, H), jnp.float32),
    )(x, msgp, degp, W_z, W_r, W_h, b_z, b_r, b_h, g, b)


def kernel(x, edge_index, edge_weights, W_z, b_z, W_r, b_r, W_h, b_h,
           ln_gamma, ln_beta):
    src = edge_index[0].astype(jnp.int32)
    dst = edge_index[1].astype(jnp.int32)
    E = src.shape[0]
    EP = NC * NS * PT
    pad = EP - E
    src_p = jnp.concatenate([src, jnp.zeros((pad,), jnp.int32)])
    dst_p = jnp.concatenate([dst, jnp.full((pad,), N, jnp.int32)])
    w_p = jnp.concatenate([edge_weights.astype(jnp.float32),
                           jnp.zeros((pad,), jnp.float32)])
    ed_t = jnp.stack([src_p.reshape(NC, NS, KCH, CHUNK),
                      dst_p.reshape(NC, NS, KCH, CHUNK)], axis=3)
    w_t = jnp.broadcast_to(w_p[:, None], (EP, 16)).reshape(NC, NS, KCH, CHUNK, 16)
    msgp, degp = _sc_aggregate(x, ed_t, w_t)
    r2 = lambda v: v.reshape(1, H)
    return _tc_gru(x, msgp, degp, W_z, W_r, W_h,
                   r2(b_z), r2(b_r), r2(b_h), r2(ln_gamma), r2(ln_beta))


# speedup vs baseline: 2.0880x; 1.2962x over previous
import functools

import jax
import jax.numpy as jnp
from jax import lax
from jax.experimental import pallas as pl
from jax.experimental.pallas import tpu as pltpu
from jax.experimental.pallas import tpu_sc as plsc

H = 128
N = 10000
NPAD = 10240
NC, NS = 2, 16
CHUNK = 88
KCH = 114
PT = KCH * CHUNK
RPS = NPAD // NS


def _sc_body(x_hbm, ed_hbm, w_hbm, msg_hbm, deg_hbm,
             ed0, ed1, wx0, wx1, gb0, gb1,
             se0, se1, sw0, sw1, sg0, sg1, acc):
    c = lax.axis_index("c")
    s = lax.axis_index("s")
    eds = (ed0, ed1)
    wxs = (wx0, wx1)
    gbs = (gb0, gb1)
    ses = (se0, se1)
    sws = (sw0, sw1)
    sgs = (sg0, sg1)

    def zrow(i, carry):
        for t in range(H // 16):
            gb1[i, pl.ds(16 * t, 16)] = jnp.zeros((16,), jnp.float32)
        return carry
    lax.fori_loop(0, CHUNK, zrow, 0)
    base = s * RPS
    def zcp(k, carry):
        pltpu.sync_copy(gb1.at[pl.ds(0, 64)], acc.at[pl.ds(base + k * 64, 64)])
        return carry
    lax.fori_loop(0, RPS // 64, zcp, 0)
    plsc.subcore_barrier()

    pltpu.sync_copy(ed_hbm.at[c, s, 0], ed0)
    pltpu.sync_copy(w_hbm.at[c, s, 0], wx0)
    pltpu.async_copy(x_hbm.at[ed0.at[0]], gb0, sg0)

    def phase1(jj, carry):
        for p in range(2):
            j = jj * 2 + p
            q = 1 - p
            jn = j + 1
            @pl.when(jn < KCH)
            def _():
                pltpu.async_copy(ed_hbm.at[c, s, jn], eds[q], ses[q])
                pltpu.async_copy(w_hbm.at[c, s, jn], wxs[q], sws[q])
            pltpu.make_async_copy(x_hbm.at[eds[p].at[0]], gbs[p],
                                  sgs[p]).wait()

            def rowgrp(g, rc):
                w16 = wxs[p][pl.ds(g * 16, 16)]
                for r in range(16):
                    i = g * 16 + r
                    wv = jnp.full((16,), w16[r], jnp.float32)
                    for t in range(H // 16):
                        gbs[p][i, pl.ds(16 * t, 16)] = (
                            gbs[p][i, pl.ds(16 * t, 16)] * wv)
                return rc
            lax.fori_loop(0, CHUNK // 16, rowgrp, 0)

            @pl.when(jn < KCH)
            def _():
                pltpu.make_async_copy(ed_hbm.at[c, s, jn], eds[q],
                                      ses[q]).wait()
                pltpu.make_async_copy(w_hbm.at[c, s, jn], wxs[q],
                                      sws[q]).wait()
                pltpu.async_copy(x_hbm.at[eds[q].at[0]], gbs[q], sgs[q])
            pltpu.sync_copy(gbs[p], acc.at[eds[p].at[1]], add=True)
        return carry
    lax.fori_loop(0, KCH // 2, phase1, 0)
    plsc.subcore_barrier()

    def zrow2(i, carry):
        for t in range(H // 16):
            gb1[i, pl.ds(16 * t, 16)] = jnp.zeros((16,), jnp.float32)
        return carry
    lax.fori_loop(0, CHUNK, zrow2, 0)
    def wb(k, carry):
        r = base + k * 64
        pltpu.sync_copy(acc.at[pl.ds(r, 64)], gb0.at[pl.ds(0, 64)])
        pltpu.sync_copy(gb0.at[pl.ds(0, 64)], msg_hbm.at[c, pl.ds(r, 64)])
        pltpu.sync_copy(gb1.at[pl.ds(0, 64)], acc.at[pl.ds(r, 64)])
        return carry
    lax.fori_loop(0, RPS // 64, wb, 0)

    def orow(i, carry):
        for t in range(H // 16):
            gb0[i, pl.ds(16 * t, 16)] = jnp.ones((16,), jnp.float32)
        return carry
    lax.fori_loop(0, CHUNK, orow, 0)
    plsc.subcore_barrier()

    pltpu.sync_copy(ed_hbm.at[c, s, 0], ed0)
    def phase2(jj, carry):
        for p in range(2):
            j = jj * 2 + p
            q = 1 - p
            jn = j + 1
            @pl.when(jn < KCH)
            def _():
                pltpu.async_copy(ed_hbm.at[c, s, jn], eds[q], ses[q])
            @pl.when(j > 0)
            def _():
                pltpu.make_async_copy(ed_hbm.at[c, s, 0], eds[p],
                                      ses[p]).wait()
            pltpu.sync_copy(gb0, acc.at[eds[p].at[1]], add=True)
        return carry
    lax.fori_loop(0, KCH // 2, phase2, 0)
    plsc.subcore_barrier()

    def wbd(k, carry):
        r = base + k * 64
        pltpu.sync_copy(acc.at[pl.ds(r, 64)], gb1.at[pl.ds(0, 64)])
        pltpu.sync_copy(gb1.at[pl.ds(0, 64)], deg_hbm.at[c, pl.ds(r, 64)])
        return carry
    lax.fori_loop(0, RPS // 64, wbd, 0)


_sc_aggregate = functools.partial(
    pl.kernel,
    mesh=plsc.VectorSubcoreMesh(core_axis_name="c", subcore_axis_name="s"),
    out_type=[
        jax.ShapeDtypeStruct((NC, NPAD, H), jnp.float32),
        jax.ShapeDtypeStruct((NC, NPAD, H), jnp.float32),
    ],
    scratch_types=(
        [pltpu.VMEM((2, CHUNK), jnp.int32) for _ in range(2)]
        + [pltpu.VMEM((CHUNK,), jnp.float32) for _ in range(2)]
        + [pltpu.VMEM((CHUNK, H), jnp.float32) for _ in range(2)]
        + [pltpu.SemaphoreType.DMA for _ in range(6)]
        + [pltpu.VMEM_SHARED((NPAD, H), jnp.float32)]
    ),
)(_sc_body)


def _tc_body(x_ref, m_ref, d_ref, wz_ref, wr_ref, wh_ref,
             bz_ref, br_ref, bh_ref, g_ref, bt_ref, o_ref):
    m = m_ref[0] + m_ref[1]
    deg = d_ref[0, :, 0:1] + d_ref[1, :, 0:1]
    deg = jnp.maximum(deg, 1.0)
    msg = m / deg
    xb = x_ref[...]

    def lin(wref, bref, a, b):
        w = wref[...]
        return (jnp.dot(a, w[:H], preferred_element_type=jnp.float32)
                + jnp.dot(b, w[H:], preferred_element_type=jnp.float32)
                + bref[...])

    z = jax.nn.sigmoid(lin(wz_ref, bz_ref, xb, msg))
    r = jax.nn.sigmoid(lin(wr_ref, br_ref, xb, msg))
    ht = jnp.tanh(lin(wh_ref, bh_ref, r * xb, msg))
    xn = (1.0 - z) * xb + z * ht
    mu = jnp.mean(xn, axis=-1, keepdims=True)
    var = jnp.mean((xn - mu) ** 2, axis=-1, keepdims=True)
    o_ref[...] = (xn - mu) * lax.rsqrt(var + 1e-5) * g_ref[...] + bt_ref[...]


def _tc_gru(x, msgp, degp, W_z, W_r, W_h, b_z, b_r, b_h, g, b):
    B = 2000
    grid = (N // B,)
    full = lambda shape: pl.BlockSpec(shape, lambda i: (0,) * len(shape))
    return pl.pallas_call(
        _tc_body,
        grid=grid,
        in_specs=[
            pl.BlockSpec((B, H), lambda i: (i, 0)),
            pl.BlockSpec((NC, B, H), lambda i: (0, i, 0)),
            pl.BlockSpec((NC, B, H), lambda i: (0, i, 0)),
            full((2 * H, H)), full((2 * H, H)), full((2 * H, H)),
            full((1, H)), full((1, H)), full((1, H)),
            full((1, H)), full((1, H)),
        ],
        out_specs=pl.BlockSpec((B, H), lambda i: (i, 0)),
        out_shape=jax.ShapeDtypeStruct((N, H), jnp.float32),
    )(x, msgp, degp, W_z, W_r, W_h, b_z, b_r, b_h, g, b)


def kernel(x, edge_index, edge_weights, W_z, b_z, W_r, b_r, W_h, b_h,
           ln_gamma, ln_beta):
    src = edge_index[0].astype(jnp.int32)
    dst = edge_index[1].astype(jnp.int32)
    E = src.shape[0]
    EP = NC * NS * PT
    pad = EP - E
    src_p = jnp.concatenate([src, jnp.zeros((pad,), jnp.int32)])
    dst_p = jnp.concatenate([dst, jnp.full((pad,), N, jnp.int32)])
    w_p = jnp.concatenate([edge_weights.astype(jnp.float32),
                           jnp.zeros((pad,), jnp.float32)])
    dst_t = dst_p.reshape(NC, NS, KCH, CHUNK)
    ed_t = jnp.stack([src_p.reshape(NC, NS, KCH, CHUNK), dst_t], axis=3)
    w_t = w_p.reshape(NC, NS, KCH, CHUNK)
    msgp, degp = _sc_aggregate(x, ed_t, w_t)
    r2 = lambda v: v.reshape(1, H)
    return _tc_gru(x, msgp, degp, W_z, W_r, W_h,
                   r2(b_z), r2(b_r), r2(b_h), r2(ln_gamma), r2(ln_beta))
